# baseline (speedup 1.0000x reference)
import functools

import jax
import jax.numpy as jnp
from jax import lax
from jax.experimental import pallas as pl
from jax.experimental.pallas import tpu as pltpu
from jax.experimental.pallas import tpu_sc as plsc

N = 10000
E = 320000
D = 128
H = D // 2
H2 = H // 2

NC = 2
NS = 16
NW = NC * NS

CHUNK = 128
NCHUNK = E // CHUNK
DEG_CPW = NCHUNK // NW
DEG_REM = NCHUNK - NW * DEG_CPW
NFULL = NCHUNK // NS
EDGE_REM = NCHUNK - NS * NFULL
DEG_GRP = 13
GRP = 3
NG = NFULL // GRP

STRIPE = 624
QSTRIPE = STRIPE // 4
TAIL = N - NS * STRIPE

_mesh = plsc.VectorSubcoreMesh(core_axis_name="c", subcore_axis_name="s")
_sc_params = pltpu.CompilerParams(use_tc_tiling_on_sc=False)


@functools.partial(
    pl.kernel,
    mesh=_mesh,
    out_type=jax.ShapeDtypeStruct((NC * N,), jnp.float32),
    scratch_types=[
        pltpu.VMEM((DEG_CPW, CHUNK), jnp.int32),
        pltpu.VMEM((CHUNK,), jnp.int32),
        pltpu.VMEM((CHUNK,), jnp.float32),
        pltpu.VMEM((STRIPE,), jnp.float32),
        pltpu.VMEM_SHARED((N,), jnp.float32),
        pltpu.SemaphoreType.DMA,
    ],
    compiler_params=_sc_params,
)
def _deg_kernel(e3_hbm, degp_hbm, idx_all, idx_t, ones_v, buf_v, deg_sh, sem):
    c = lax.axis_index("c")
    s = lax.axis_index("s")
    w = s * NC + c
    for j in range(STRIPE // 16):
        buf_v[pl.ds(j * 16, 16)] = jnp.zeros((16,), jnp.float32)
    pltpu.sync_copy(buf_v, deg_sh.at[pl.ds(s * STRIPE, STRIPE)])
    @pl.when(s == 0)
    def _():
        pltpu.sync_copy(buf_v.at[pl.ds(0, TAIL)],
                        deg_sh.at[pl.ds(NS * STRIPE, TAIL)])
    for j in range(CHUNK // 16):
        ones_v[pl.ds(j * 16, 16)] = jnp.ones((16,), jnp.float32)
    plsc.subcore_barrier()

    pltpu.sync_copy(e3_hbm.at[0, pl.ds(w * DEG_CPW, DEG_CPW)], idx_all)

    def group(g, carry):
        for j in range(DEG_GRP):
            pltpu.async_copy(ones_v, deg_sh.at[idx_all.at[g * DEG_GRP + j]],
                             sem, add=True)
        for j in range(DEG_GRP):
            pltpu.make_async_copy(ones_v,
                                  deg_sh.at[idx_all.at[g * DEG_GRP + j]],
                                  sem).wait()
        return carry

    lax.fori_loop(0, DEG_CPW // DEG_GRP, group, 0)
    @pl.when(w < DEG_REM)
    def _():
        pltpu.sync_copy(e3_hbm.at[0, NW * DEG_CPW + w], idx_t)
        pltpu.sync_copy(ones_v, deg_sh.at[idx_t], add=True)
    plsc.subcore_barrier()
    pltpu.sync_copy(deg_sh.at[pl.ds(s * STRIPE, STRIPE)], buf_v)
    pltpu.sync_copy(buf_v, degp_hbm.at[pl.ds(c * N + s * STRIPE, STRIPE)])
    @pl.when(s == 0)
    def _():
        pltpu.sync_copy(deg_sh.at[pl.ds(NS * STRIPE, TAIL)],
                        buf_v.at[pl.ds(0, TAIL)])
        pltpu.sync_copy(buf_v.at[pl.ds(0, TAIL)],
                        degp_hbm.at[pl.ds(c * N + NS * STRIPE, TAIL)])


@functools.partial(
    pl.kernel,
    mesh=_mesh,
    out_type=jax.ShapeDtypeStruct((NC, N, H), jnp.float32),
    scratch_types=[
        pltpu.VMEM((NFULL, CHUNK), jnp.int32),
        pltpu.VMEM((NFULL, CHUNK), jnp.int32),
        pltpu.VMEM((GRP, CHUNK, H2), jnp.float32),
        pltpu.VMEM((GRP, CHUNK, H2), jnp.float32),
        pltpu.VMEM((CHUNK,), jnp.int32),
        pltpu.VMEM((CHUNK,), jnp.int32),
        pltpu.VMEM((QSTRIPE, H2), jnp.float32),
        pltpu.VMEM_SHARED((N, H2), jnp.float32),
        pltpu.VMEM_SHARED((N, H2), jnp.float32),
        pltpu.SemaphoreType.DMA,
        pltpu.SemaphoreType.DMA,
        pltpu.SemaphoreType.DMA,
        pltpu.SemaphoreType.DMA,
    ],
    compiler_params=_sc_params,
)
def _edge_kernel(e3_hbm, y1_hbm, y2_hbm, z2_hbm, outp_hbm,
                 sidx_all, didx_all, rows_a, rows_b, sidx_t, didx_t,
                 buf_v, y_sh, acc_sh, sem_ga, sem_gb, sem_sa, sem_sb):
    c = lax.axis_index("c")
    s = lax.axis_index("s")

    pltpu.sync_copy(e3_hbm.at[1, pl.ds(s * NFULL, NFULL)], sidx_all)
    pltpu.sync_copy(e3_hbm.at[0, pl.ds(s * NFULL, NFULL)], didx_all)

    for r in range(2):
        pltpu.sync_copy(z2_hbm, buf_v)
        for k in range(4):
            pltpu.sync_copy(
                buf_v, acc_sh.at[pl.ds(s * STRIPE + k * QSTRIPE, QSTRIPE)])
        @pl.when(s == 0)
        def _():
            pltpu.sync_copy(buf_v.at[pl.ds(0, TAIL)],
                            acc_sh.at[pl.ds(NS * STRIPE, TAIL)])
        for k in range(4):
            row0 = s * STRIPE + k * QSTRIPE
            @pl.when(c == 0)
            def _():
                pltpu.sync_copy(
                    y1_hbm.at[pl.ds(row0, QSTRIPE), pl.ds(r * H2, H2)], buf_v)
            @pl.when(c == 1)
            def _():
                pltpu.sync_copy(
                    y2_hbm.at[pl.ds(row0, QSTRIPE), pl.ds(r * H2, H2)], buf_v)
            pltpu.sync_copy(buf_v, y_sh.at[pl.ds(row0, QSTRIPE)])
        @pl.when(s == 0)
        def _():
            @pl.when(c == 0)
            def _():
                pltpu.sync_copy(
                    y1_hbm.at[pl.ds(NS * STRIPE, TAIL), pl.ds(r * H2, H2)],
                    buf_v.at[pl.ds(0, TAIL)])
            @pl.when(c == 1)
            def _():
                pltpu.sync_copy(
                    y2_hbm.at[pl.ds(NS * STRIPE, TAIL), pl.ds(r * H2, H2)],
                    buf_v.at[pl.ds(0, TAIL)])
            pltpu.sync_copy(buf_v.at[pl.ds(0, TAIL)],
                            y_sh.at[pl.ds(NS * STRIPE, TAIL)])
        plsc.subcore_barrier()

        def fire_gathers(g, rows, sem):
            for j in range(GRP):
                pltpu.async_copy(y_sh.at[sidx_all.at[g * GRP + j]],
                                 rows.at[j], sem)

        def drain_gather(g, j, rows, sem):
            pltpu.make_async_copy(y_sh.at[sidx_all.at[g * GRP + j]],
                                  rows.at[j], sem).wait()

        def fire_scatter(g, j, rows, sem):
            pltpu.async_copy(rows.at[j],
                             acc_sh.at[didx_all.at[g * GRP + j]],
                             sem, add=True)

        def drain_scatters(g, rows, sem):
            for j in range(GRP):
                pltpu.make_async_copy(rows.at[j],
                                      acc_sh.at[didx_all.at[g * GRP + j]],
                                      sem).wait()

        def step(g, rows_c, sem_gc, sem_sc, rows_n, sem_gn, sem_sn):
            @pl.when(g + 1 < NG)
            def _():
                @pl.when(g >= 1)
                def _():
                    drain_scatters(g - 1, rows_n, sem_sn)
                fire_gathers(g + 1, rows_n, sem_gn)
            for j in range(GRP):
                drain_gather(g, j, rows_c, sem_gc)
                fire_scatter(g, j, rows_c, sem_sc)

        fire_gathers(0, rows_a, sem_ga)

        def body(p, carry):
            step(2 * p, rows_a, sem_ga, sem_sa, rows_b, sem_gb, sem_sb)
            step(2 * p + 1, rows_b, sem_gb, sem_sb, rows_a, sem_ga, sem_sa)
            return carry

        lax.fori_loop(0, NG // 2, body, 0)
        drain_scatters(NG - 2, rows_a, sem_sa)
        drain_scatters(NG - 1, rows_b, sem_sb)

        @pl.when(s < EDGE_REM)
        def _():
            ci = NS * NFULL + s
            pltpu.sync_copy(e3_hbm.at[1, ci], sidx_t)
            pltpu.sync_copy(e3_hbm.at[0, ci], didx_t)
            pltpu.sync_copy(y_sh.at[sidx_t], rows_a.at[0])
            pltpu.sync_copy(rows_a.at[0], acc_sh.at[didx_t], add=True)
        plsc.subcore_barrier()
        for k in range(4):
            row0 = s * STRIPE + k * QSTRIPE
            pltpu.sync_copy(acc_sh.at[pl.ds(row0, QSTRIPE)], buf_v)
            pltpu.sync_copy(
                buf_v,
                outp_hbm.at[c, pl.ds(row0, QSTRIPE), pl.ds(r * H2, H2)])
        @pl.when(s == 0)
        def _():
            pltpu.sync_copy(acc_sh.at[pl.ds(NS * STRIPE, TAIL)],
                            buf_v.at[pl.ds(0, TAIL)])
            pltpu.sync_copy(
                buf_v.at[pl.ds(0, TAIL)],
                outp_hbm.at[c, pl.ds(NS * STRIPE, TAIL), pl.ds(r * H2, H2)])
        plsc.subcore_barrier()


ROWS_B = 2000


def _y_body(x_ref, w_ref, degp_ref, y1_ref, y2_ref):
    deg = jnp.maximum(degp_ref[0] + degp_ref[1], 1.0)
    s = lax.rsqrt(deg)
    y = jnp.dot(x_ref[...], w_ref[...],
                preferred_element_type=jnp.float32) * s
    y1_ref[...] = y[:, :H]
    y2_ref[...] = y[:, H:]


def _y_call(x, W, degp3):
    return pl.pallas_call(
        _y_body,
        grid=(N // ROWS_B,),
        in_specs=[
            pl.BlockSpec((ROWS_B, D), lambda i: (i, 0)),
            pl.BlockSpec((D, D), lambda i: (0, 0)),
            pl.BlockSpec((NC, ROWS_B, 1), lambda i: (0, i, 0)),
        ],
        out_specs=[
            pl.BlockSpec((ROWS_B, H), lambda i: (i, 0)),
            pl.BlockSpec((ROWS_B, H), lambda i: (i, 0)),
        ],
        out_shape=[
            jax.ShapeDtypeStruct((N, H), jnp.float32),
            jax.ShapeDtypeStruct((N, H), jnp.float32),
        ],
    )(x, W, degp3)


def _out_body(outp_ref, degp_ref, o_ref):
    deg = jnp.maximum(degp_ref[0] + degp_ref[1], 1.0)
    s = lax.rsqrt(deg)
    o_ref[...] = jnp.concatenate([outp_ref[0], outp_ref[1]], axis=-1) * s


def _out_call(outp, degp3):
    return pl.pallas_call(
        _out_body,
        grid=(N // ROWS_B,),
        in_specs=[
            pl.BlockSpec((NC, ROWS_B, H), lambda i: (0, i, 0)),
            pl.BlockSpec((NC, ROWS_B, 1), lambda i: (0, i, 0)),
        ],
        out_specs=pl.BlockSpec((ROWS_B, D), lambda i: (i, 0)),
        out_shape=jax.ShapeDtypeStruct((N, D), jnp.float32),
    )(outp, degp3)


def kernel(x, edge_index, W):
    e3 = edge_index.reshape(2, NCHUNK, CHUNK)
    z2 = jnp.zeros((QSTRIPE, H2), jnp.float32)
    degp = _deg_kernel(e3)
    degp3 = degp.reshape(NC, N, 1)
    y1, y2 = _y_call(x, W, degp3)
    outp = _edge_kernel(e3, y1, y2, z2)
    return _out_call(outp, degp3)


# speedup vs baseline: 1.1328x; 1.1328x over previous
import functools

import jax
import jax.numpy as jnp
from jax import lax
from jax.experimental import pallas as pl
from jax.experimental.pallas import tpu as pltpu
from jax.experimental.pallas import tpu_sc as plsc

N = 10000
E = 320000
D = 128
H = D // 2
H2 = H // 2

NC = 2
NS = 16
NW = NC * NS

CHUNK = 128
NCHUNK = E // CHUNK
DEG_CPW = NCHUNK // NW
DEG_REM = NCHUNK - NW * DEG_CPW
NFULL = NCHUNK // NS
EDGE_REM = NCHUNK - NS * NFULL
DEG_GRP = 13
EGRP = 12

STRIPE = 624
QSTRIPE = STRIPE // 4
TAIL = N - NS * STRIPE

_mesh = plsc.VectorSubcoreMesh(core_axis_name="c", subcore_axis_name="s")
_sc_params = pltpu.CompilerParams(use_tc_tiling_on_sc=False)


@functools.partial(
    pl.kernel,
    mesh=_mesh,
    out_type=jax.ShapeDtypeStruct((NC * N,), jnp.float32),
    scratch_types=[
        pltpu.VMEM((DEG_CPW, CHUNK), jnp.int32),
        pltpu.VMEM((CHUNK,), jnp.int32),
        pltpu.VMEM((CHUNK,), jnp.float32),
        pltpu.VMEM((STRIPE,), jnp.float32),
        pltpu.VMEM_SHARED((N,), jnp.float32),
        pltpu.SemaphoreType.DMA,
    ],
    compiler_params=_sc_params,
)
def _deg_kernel(e3_hbm, degp_hbm, idx_all, idx_t, ones_v, buf_v, deg_sh, sem):
    c = lax.axis_index("c")
    s = lax.axis_index("s")
    w = s * NC + c
    for j in range(STRIPE // 16):
        buf_v[pl.ds(j * 16, 16)] = jnp.zeros((16,), jnp.float32)
    pltpu.sync_copy(buf_v, deg_sh.at[pl.ds(s * STRIPE, STRIPE)])
    @pl.when(s == 0)
    def _():
        pltpu.sync_copy(buf_v.at[pl.ds(0, TAIL)],
                        deg_sh.at[pl.ds(NS * STRIPE, TAIL)])
    for j in range(CHUNK // 16):
        ones_v[pl.ds(j * 16, 16)] = jnp.ones((16,), jnp.float32)
    plsc.subcore_barrier()

    pltpu.sync_copy(e3_hbm.at[0, pl.ds(w * DEG_CPW, DEG_CPW)], idx_all)

    def group(g, carry):
        for j in range(DEG_GRP):
            pltpu.async_copy(ones_v, deg_sh.at[idx_all.at[g * DEG_GRP + j]],
                             sem, add=True)
        for j in range(DEG_GRP):
            pltpu.make_async_copy(ones_v,
                                  deg_sh.at[idx_all.at[g * DEG_GRP + j]],
                                  sem).wait()
        return carry

    lax.fori_loop(0, DEG_CPW // DEG_GRP, group, 0)
    @pl.when(w < DEG_REM)
    def _():
        pltpu.sync_copy(e3_hbm.at[0, NW * DEG_CPW + w], idx_t)
        pltpu.sync_copy(ones_v, deg_sh.at[idx_t], add=True)
    plsc.subcore_barrier()
    pltpu.sync_copy(deg_sh.at[pl.ds(s * STRIPE, STRIPE)], buf_v)
    pltpu.sync_copy(buf_v, degp_hbm.at[pl.ds(c * N + s * STRIPE, STRIPE)])
    @pl.when(s == 0)
    def _():
        pltpu.sync_copy(deg_sh.at[pl.ds(NS * STRIPE, TAIL)],
                        buf_v.at[pl.ds(0, TAIL)])
        pltpu.sync_copy(buf_v.at[pl.ds(0, TAIL)],
                        degp_hbm.at[pl.ds(c * N + NS * STRIPE, TAIL)])


@functools.partial(
    pl.kernel,
    mesh=_mesh,
    out_type=jax.ShapeDtypeStruct((NC, N, H), jnp.float32),
    scratch_types=[
        pltpu.VMEM((NFULL, CHUNK), jnp.int32),
        pltpu.VMEM((NFULL, CHUNK), jnp.int32),
        pltpu.VMEM((CHUNK, H), jnp.float32),
        pltpu.VMEM((CHUNK, H), jnp.float32),
        pltpu.VMEM((CHUNK,), jnp.int32),
        pltpu.VMEM((CHUNK,), jnp.int32),
        pltpu.VMEM((QSTRIPE, H), jnp.float32),
        pltpu.VMEM_SHARED((N, H), jnp.float32),
        pltpu.SemaphoreType.DMA,
        pltpu.SemaphoreType.DMA,
        pltpu.SemaphoreType.DMA,
        pltpu.SemaphoreType.DMA,
    ],
    compiler_params=_sc_params,
)
def _edge_kernel(e3_hbm, y1_hbm, y2_hbm, z2_hbm, outp_hbm,
                 sidx_all, didx_all, rows_a, rows_b, sidx_t, didx_t,
                 buf_v, acc_sh, sem_ga, sem_gb, sem_sa, sem_sb):
    c = lax.axis_index("c")
    s = lax.axis_index("s")
    pltpu.sync_copy(z2_hbm, buf_v)
    for k in range(4):
        pltpu.sync_copy(buf_v,
                        acc_sh.at[pl.ds(s * STRIPE + k * QSTRIPE, QSTRIPE)])
    @pl.when(s == 0)
    def _():
        pltpu.sync_copy(buf_v.at[pl.ds(0, TAIL)],
                        acc_sh.at[pl.ds(NS * STRIPE, TAIL)])

    pltpu.sync_copy(e3_hbm.at[1, pl.ds(s * NFULL, NFULL)], sidx_all)
    pltpu.sync_copy(e3_hbm.at[0, pl.ds(s * NFULL, NFULL)], didx_all)
    plsc.subcore_barrier()

    def gather(sidx, rows, sem):
        @pl.when(c == 0)
        def _():
            pltpu.async_copy(y1_hbm.at[sidx], rows, sem)
        @pl.when(c == 1)
        def _():
            pltpu.async_copy(y2_hbm.at[sidx], rows, sem)

    def gather_wait(sidx, rows, sem):
        @pl.when(c == 0)
        def _():
            pltpu.make_async_copy(y1_hbm.at[sidx], rows, sem).wait()
        @pl.when(c == 1)
        def _():
            pltpu.make_async_copy(y2_hbm.at[sidx], rows, sem).wait()

    def step(i, rows_c, sem_gc, sem_sc, rows_n, sem_gn, sem_sn):
        @pl.when(i + 1 < NFULL)
        def _():
            @pl.when(i >= 1)
            def _():
                pltpu.make_async_copy(rows_n, acc_sh.at[didx_all.at[i - 1]],
                                      sem_sn).wait()
            gather(sidx_all.at[i + 1], rows_n, sem_gn)
        gather_wait(sidx_all.at[i], rows_c, sem_gc)
        pltpu.async_copy(rows_c, acc_sh.at[didx_all.at[i]], sem_sc, add=True)

    gather(sidx_all.at[0], rows_a, sem_ga)

    def body(p, carry):
        step(2 * p, rows_a, sem_ga, sem_sa, rows_b, sem_gb, sem_sb)
        step(2 * p + 1, rows_b, sem_gb, sem_sb, rows_a, sem_ga, sem_sa)
        return carry

    lax.fori_loop(0, NFULL // 2, body, 0)
    pltpu.make_async_copy(rows_a, acc_sh.at[didx_all.at[NFULL - 2]],
                          sem_sa).wait()
    pltpu.make_async_copy(rows_b, acc_sh.at[didx_all.at[NFULL - 1]],
                          sem_sb).wait()

    @pl.when(s < EDGE_REM)
    def _():
        ci = NS * NFULL + s
        pltpu.sync_copy(e3_hbm.at[1, ci], sidx_t)
        pltpu.sync_copy(e3_hbm.at[0, ci], didx_t)
        gather(sidx_t, rows_a, sem_ga)
        gather_wait(sidx_t, rows_a, sem_ga)
        pltpu.sync_copy(rows_a, acc_sh.at[didx_t], add=True)
    plsc.subcore_barrier()
    for k in range(4):
        pltpu.sync_copy(
            acc_sh.at[pl.ds(s * STRIPE + k * QSTRIPE, QSTRIPE)], buf_v)
        pltpu.sync_copy(
            buf_v, outp_hbm.at[c, pl.ds(s * STRIPE + k * QSTRIPE, QSTRIPE)])
    @pl.when(s == 0)
    def _():
        pltpu.sync_copy(acc_sh.at[pl.ds(NS * STRIPE, TAIL)],
                        buf_v.at[pl.ds(0, TAIL)])
        pltpu.sync_copy(buf_v.at[pl.ds(0, TAIL)],
                        outp_hbm.at[c, pl.ds(NS * STRIPE, TAIL)])


ROWS_B = 2000


def _y_body(x_ref, w_ref, degp_ref, y1_ref, y2_ref):
    deg = jnp.maximum(degp_ref[0] + degp_ref[1], 1.0)
    s = lax.rsqrt(deg)
    y = jnp.dot(x_ref[...], w_ref[...],
                preferred_element_type=jnp.float32) * s
    y1_ref[...] = y[:, :H]
    y2_ref[...] = y[:, H:]


def _y_call(x, W, degp3):
    return pl.pallas_call(
        _y_body,
        grid=(N // ROWS_B,),
        in_specs=[
            pl.BlockSpec((ROWS_B, D), lambda i: (i, 0)),
            pl.BlockSpec((D, D), lambda i: (0, 0)),
            pl.BlockSpec((NC, ROWS_B, 1), lambda i: (0, i, 0)),
        ],
        out_specs=[
            pl.BlockSpec((ROWS_B, H), lambda i: (i, 0)),
            pl.BlockSpec((ROWS_B, H), lambda i: (i, 0)),
        ],
        out_shape=[
            jax.ShapeDtypeStruct((N, H), jnp.float32),
            jax.ShapeDtypeStruct((N, H), jnp.float32),
        ],
    )(x, W, degp3)


def _out_body(outp_ref, degp_ref, o_ref):
    deg = jnp.maximum(degp_ref[0] + degp_ref[1], 1.0)
    s = lax.rsqrt(deg)
    o_ref[...] = jnp.concatenate([outp_ref[0], outp_ref[1]], axis=-1) * s


def _out_call(outp, degp3):
    return pl.pallas_call(
        _out_body,
        grid=(N // ROWS_B,),
        in_specs=[
            pl.BlockSpec((NC, ROWS_B, H), lambda i: (0, i, 0)),
            pl.BlockSpec((NC, ROWS_B, 1), lambda i: (0, i, 0)),
        ],
        out_specs=pl.BlockSpec((ROWS_B, D), lambda i: (i, 0)),
        out_shape=jax.ShapeDtypeStruct((N, D), jnp.float32),
    )(outp, degp3)


def kernel(x, edge_index, W):
    e3 = edge_index.reshape(2, NCHUNK, CHUNK)
    z2 = jnp.zeros((QSTRIPE, H), jnp.float32)
    degp = _deg_kernel(e3)
    degp3 = degp.reshape(NC, N, 1)
    y1, y2 = _y_call(x, W, degp3)
    outp = _edge_kernel(e3, y1, y2, z2)
    return _out_call(outp, degp3)
